# baseline (speedup 1.0000x reference)
import jax
import jax.numpy as jnp
from jax.experimental import pallas as pl
from jax.experimental.pallas import tpu as pltpu

_EPS = 0.001
_CLIP_LO = 1e-07
_CLIP_HI = 10000000.0

_ROWS_PER_BLOCK = 1536


def _loss_block_kernel(p_ref, t_ref, s0_ref, s1_ref, n0_ref):
    @pl.when(pl.program_id(1) == 0)
    def _init():
        s0_ref[...] = jnp.zeros((1, 1, 1), jnp.float32)
        s1_ref[...] = jnp.zeros((1, 1, 1), jnp.float32)
        n0_ref[...] = jnp.zeros((1, 1, 1), jnp.int32)

    p = p_ref[...]
    t = t_ref[...]
    e = p - t
    s = jnp.clip(e * e, _CLIP_LO, _CLIP_HI)
    m0 = t > _EPS
    s0_ref[...] += jnp.sum(jnp.where(m0, s, 0.0)).reshape(1, 1, 1)
    s1_ref[...] += jnp.sum(jnp.where(m0, 0.0, s)).reshape(1, 1, 1)
    n0_ref[...] += jnp.sum(m0.astype(jnp.int32)).reshape(1, 1, 1)


def kernel(pred, truth):
    n_total = pred.size
    p2 = pred.reshape(-1, pred.shape[-1])
    t2 = truth.reshape(-1, truth.shape[-1])
    rows, cols = p2.shape
    n_cores = 2
    blocks_per_core = rows // (_ROWS_PER_BLOCK * n_cores)

    in_spec = pl.BlockSpec(
        (_ROWS_PER_BLOCK, cols),
        lambda i, j, nb=blocks_per_core: (i * nb + j, 0),
    )
    out_spec = pl.BlockSpec((1, 1, 1), lambda i, j: (i, 0, 0))

    s0, s1, n0 = pl.pallas_call(
        _loss_block_kernel,
        grid=(n_cores, blocks_per_core),
        in_specs=[in_spec, in_spec],
        out_specs=[out_spec, out_spec, out_spec],
        out_shape=[
            jax.ShapeDtypeStruct((n_cores, 1, 1), jnp.float32),
            jax.ShapeDtypeStruct((n_cores, 1, 1), jnp.float32),
            jax.ShapeDtypeStruct((n_cores, 1, 1), jnp.int32),
        ],
        compiler_params=pltpu.CompilerParams(
            dimension_semantics=("parallel", "arbitrary"),
        ),
    )(p2, t2)

    s0 = jnp.sum(s0)
    s1 = jnp.sum(s1)
    n0 = jnp.sum(n0).astype(jnp.float32)
    n1 = jnp.float32(n_total) - n0
    mean1 = s1 / jnp.maximum(n1, 1.0)
    mean0 = jnp.where(n0 > 0, s0 / jnp.maximum(n0, 1.0), mean1)
    return mean0 + mean1


# speedup vs baseline: 1.1815x; 1.1815x over previous
import jax
import jax.numpy as jnp
from jax.experimental import pallas as pl
from jax.experimental.pallas import tpu as pltpu

_EPS = 0.001
_CLIP_LO = 1e-07
_CLIP_HI = 10000000.0

_ROWS_PER_BLOCK = 3072


def _loss_block_kernel(p_ref, t_ref, s0_ref, s1_ref, n0_ref):
    @pl.when(pl.program_id(1) == 0)
    def _init():
        s0_ref[...] = jnp.zeros((1, 1, 1), jnp.float32)
        s1_ref[...] = jnp.zeros((1, 1, 1), jnp.float32)
        n0_ref[...] = jnp.zeros((1, 1, 1), jnp.int32)

    p = p_ref[...]
    t = t_ref[...]
    e = p - t
    s = jnp.clip(e * e, _CLIP_LO, _CLIP_HI)
    m0 = t > _EPS
    s0_ref[...] += jnp.sum(jnp.where(m0, s, 0.0)).reshape(1, 1, 1)
    s1_ref[...] += jnp.sum(jnp.where(m0, 0.0, s)).reshape(1, 1, 1)
    n0_ref[...] += jnp.sum(m0.astype(jnp.int32)).reshape(1, 1, 1)


def kernel(pred, truth):
    n_total = pred.size
    p2 = pred.reshape(-1, pred.shape[-1])
    t2 = truth.reshape(-1, truth.shape[-1])
    rows, cols = p2.shape
    n_cores = 1
    blocks_per_core = rows // (_ROWS_PER_BLOCK * n_cores)

    in_spec = pl.BlockSpec(
        (_ROWS_PER_BLOCK, cols),
        lambda i, j, nb=blocks_per_core: (i * nb + j, 0),
    )
    out_spec = pl.BlockSpec((1, 1, 1), lambda i, j: (i, 0, 0))

    s0, s1, n0 = pl.pallas_call(
        _loss_block_kernel,
        grid=(n_cores, blocks_per_core),
        in_specs=[in_spec, in_spec],
        out_specs=[out_spec, out_spec, out_spec],
        out_shape=[
            jax.ShapeDtypeStruct((n_cores, 1, 1), jnp.float32),
            jax.ShapeDtypeStruct((n_cores, 1, 1), jnp.float32),
            jax.ShapeDtypeStruct((n_cores, 1, 1), jnp.int32),
        ],
        compiler_params=pltpu.CompilerParams(
            dimension_semantics=("parallel", "arbitrary"),
        ),
    )(p2, t2)

    s0 = jnp.sum(s0)
    s1 = jnp.sum(s1)
    n0 = jnp.sum(n0).astype(jnp.float32)
    n1 = jnp.float32(n_total) - n0
    mean1 = s1 / jnp.maximum(n1, 1.0)
    mean0 = jnp.where(n0 > 0, s0 / jnp.maximum(n0, 1.0), mean1)
    return mean0 + mean1


# speedup vs baseline: 1.2664x; 1.0718x over previous
import jax
import jax.numpy as jnp
from jax.experimental import pallas as pl
from jax.experimental.pallas import tpu as pltpu

_EPS = 0.001
_CLIP_LO = 1e-07
_CLIP_HI = 10000000.0

_ROWS_PER_BLOCK = 6144


def _loss_block_kernel(p_ref, t_ref, s0_ref, s1_ref, n0_ref):
    @pl.when(pl.program_id(1) == 0)
    def _init():
        s0_ref[...] = jnp.zeros((1, 1, 1), jnp.float32)
        s1_ref[...] = jnp.zeros((1, 1, 1), jnp.float32)
        n0_ref[...] = jnp.zeros((1, 1, 1), jnp.int32)

    p = p_ref[...]
    t = t_ref[...]
    e = p - t
    s = jnp.clip(e * e, _CLIP_LO, _CLIP_HI)
    m0 = t > _EPS
    s0_ref[...] += jnp.sum(jnp.where(m0, s, 0.0)).reshape(1, 1, 1)
    s1_ref[...] += jnp.sum(jnp.where(m0, 0.0, s)).reshape(1, 1, 1)
    n0_ref[...] += jnp.sum(m0.astype(jnp.int32)).reshape(1, 1, 1)


def kernel(pred, truth):
    n_total = pred.size
    p2 = pred.reshape(-1, pred.shape[-1])
    t2 = truth.reshape(-1, truth.shape[-1])
    rows, cols = p2.shape
    n_cores = 1
    blocks_per_core = rows // (_ROWS_PER_BLOCK * n_cores)

    in_spec = pl.BlockSpec(
        (_ROWS_PER_BLOCK, cols),
        lambda i, j, nb=blocks_per_core: (i * nb + j, 0),
    )
    out_spec = pl.BlockSpec((1, 1, 1), lambda i, j: (i, 0, 0))

    s0, s1, n0 = pl.pallas_call(
        _loss_block_kernel,
        grid=(n_cores, blocks_per_core),
        in_specs=[in_spec, in_spec],
        out_specs=[out_spec, out_spec, out_spec],
        out_shape=[
            jax.ShapeDtypeStruct((n_cores, 1, 1), jnp.float32),
            jax.ShapeDtypeStruct((n_cores, 1, 1), jnp.float32),
            jax.ShapeDtypeStruct((n_cores, 1, 1), jnp.int32),
        ],
        compiler_params=pltpu.CompilerParams(
            dimension_semantics=("parallel", "arbitrary"),
        ),
    )(p2, t2)

    s0 = jnp.sum(s0)
    s1 = jnp.sum(s1)
    n0 = jnp.sum(n0).astype(jnp.float32)
    n1 = jnp.float32(n_total) - n0
    mean1 = s1 / jnp.maximum(n1, 1.0)
    mean0 = jnp.where(n0 > 0, s0 / jnp.maximum(n0, 1.0), mean1)
    return mean0 + mean1
